# baseline (speedup 1.0000x reference)
import functools
import math

import jax
import jax.numpy as jnp
from jax import lax
from jax.experimental import pallas as pl
from jax.experimental.pallas import tpu as pltpu
from jax.experimental.pallas import tpu_sc as plsc


def _emb_call(x, table, pe_slice):
    B, L = x.shape
    V, D = table.shape
    scale = math.sqrt(D)

    info = plsc.get_sparse_core_info()
    num_cores = info.num_cores
    num_workers = info.num_cores * info.num_subcores
    assert B % num_workers == 0
    b_per_w = B // num_workers

    c0, c1 = 120, L - 120

    mesh = plsc.VectorSubcoreMesh(core_axis_name="c", subcore_axis_name="s")

    @functools.partial(
        pl.kernel,
        mesh=mesh,
        out_type=jax.ShapeDtypeStruct((B, L, D), jnp.float32),
        compiler_params=pltpu.CompilerParams(use_tc_tiling_on_sc=False),
        scratch_types=[
            pltpu.VMEM((L,), jnp.int32),
            pltpu.VMEM((L, D), jnp.float32),
            pltpu.VMEM((L, D), jnp.float32),
            pltpu.SemaphoreType.DMA,
        ],
    )
    def emb_kernel(x_hbm, table_hbm, pe_hbm, out_hbm, idx_v, rows_v, pe_v, sem):
        wid = lax.axis_index("s") * num_cores + lax.axis_index("c")
        base = wid * b_per_w
        pltpu.sync_copy(pe_hbm, pe_v)

        def body(i, carry):
            b = base + i
            pltpu.sync_copy(x_hbm.at[b], idx_v)
            cp1 = pltpu.async_copy(
                table_hbm.at[idx_v.at[pl.ds(0, c0)]], rows_v.at[pl.ds(0, c0)], sem
            )
            cp2 = pltpu.async_copy(
                table_hbm.at[idx_v.at[pl.ds(c0, c1)]], rows_v.at[pl.ds(c0, c1)], sem
            )
            cp1.wait()
            cp2.wait()

            def vbody(r, vc):
                for c in range(D // 16):
                    s = pl.ds(c * 16, 16)
                    rows_v[r, s] = rows_v[r, s] * scale + pe_v[r, s]
                return vc

            lax.fori_loop(0, L, vbody, 0)
            pltpu.sync_copy(rows_v, out_hbm.at[b])
            return carry

        lax.fori_loop(0, b_per_w, body, 0)

    return emb_kernel(x, table, pe_slice)


def kernel(x, table, pe):
    pe_slice = pe[0, : x.shape[1], :]
    return _emb_call(x, table, pe_slice)
